# baseline (speedup 1.0000x reference)
import functools

import jax
import jax.numpy as jnp
from jax.experimental import pallas as pl
from jax.experimental.pallas import tpu as pltpu

_WIN = 5
_PAD = (_WIN - 1) // 2


def _vq_loss_body(x_ref, e_ref, out_ref, *, Tlen, tblk):
    xb = x_ref[0]
    e = e_ref[...]
    e2 = jnp.sum(e * e, axis=0)
    ebf = jnp.concatenate(
        [e, (-0.5 * e2)[None, :]], axis=0
    ).astype(jnp.bfloat16)

    s_all = jnp.sum(xb * xb)
    c0 = xb[:, _PAD:_PAD + 1]
    c1 = xb[:, _PAD + 1:_PAD + 2]
    c2 = xb[:, Tlen:Tlen + 1]
    c3 = xb[:, Tlen + 1:Tlen + 2]
    corr = (2.0 * jnp.sum(c0 * c0) + jnp.sum(c1 * c1)
            + jnp.sum(c2 * c2) + 2.0 * jnp.sum(c3 * c3))
    f2 = 5.0 * s_all - corr

    xbb = xb.astype(jnp.bfloat16)
    ft = jnp.concatenate(
        [xbb[:, w:w + Tlen] for w in range(_WIN)]
        + [jnp.ones((1, Tlen), jnp.bfloat16)],
        axis=0,
    )

    acc_v = jnp.zeros((tblk,), jnp.float32)
    for t0 in range(0, Tlen, tblk):
        g = jax.lax.dot_general(
            ebf, ft[:, t0:t0 + tblk],
            dimension_numbers=(((0,), (0,)), ((), ())),
            preferred_element_type=jnp.float32,
        )
        acc_v = acc_v + jnp.max(g, axis=0)

    out_ref[0, 0, 0] = f2 - 2.0 * jnp.sum(acc_v)


@jax.jit
def kernel(x, embedding):
    B, P, T = x.shape
    K, D = embedding.shape
    xp = jnp.pad(x, ((0, 0), (0, 0), (_PAD, _PAD)))
    et = jnp.transpose(embedding.reshape(K, P, _WIN), (2, 1, 0)).reshape(D, K)

    body = functools.partial(_vq_loss_body, Tlen=T, tblk=1024)
    partials = pl.pallas_call(
        body,
        grid=(B,),
        in_specs=[
            pl.BlockSpec((1, P, T + 2 * _PAD), lambda b: (b, 0, 0)),
            pl.BlockSpec((D, K), lambda b: (0, 0)),
        ],
        out_specs=pl.BlockSpec((1, 1, 1), lambda b: (b, 0, 0), memory_space=pltpu.SMEM),
        out_shape=jax.ShapeDtypeStruct((B, 1, 1), jnp.float32),
    )(xp, et)
    total = jnp.sum(partials)
    return 0.25 * total / (B * T * D)


# speedup vs baseline: 1.1446x; 1.1446x over previous
import functools

import jax
import jax.numpy as jnp
from jax.experimental import pallas as pl
from jax.experimental.pallas import tpu as pltpu

_WIN = 5
_PAD = (_WIN - 1) // 2


def _vq_loss_body(x_ref, e_ref, out_ref, *, Tlen, tblk, nb):
    e = e_ref[...]
    e2 = jnp.sum(e * e, axis=0)
    ebf = jnp.concatenate(
        [e, (-0.5 * e2)[None, :]], axis=0
    ).astype(jnp.bfloat16)

    total = jnp.float32(0.0)
    acc_v = jnp.zeros((tblk,), jnp.float32)
    for i in range(nb):
        xb = x_ref[i]

        s_all = jnp.sum(xb * xb)
        c0 = xb[:, _PAD:_PAD + 1]
        c1 = xb[:, _PAD + 1:_PAD + 2]
        c2 = xb[:, Tlen:Tlen + 1]
        c3 = xb[:, Tlen + 1:Tlen + 2]
        corr = (2.0 * jnp.sum(c0 * c0) + jnp.sum(c1 * c1)
                + jnp.sum(c2 * c2) + 2.0 * jnp.sum(c3 * c3))
        total = total + 5.0 * s_all - corr

        xbb = xb.astype(jnp.bfloat16)
        ft = jnp.concatenate(
            [xbb[:, w:w + Tlen] for w in range(_WIN)]
            + [jnp.ones((1, Tlen), jnp.bfloat16)],
            axis=0,
        )

        for t0 in range(0, Tlen, tblk):
            g = jax.lax.dot_general(
                ebf, ft[:, t0:t0 + tblk],
                dimension_numbers=(((0,), (0,)), ((), ())),
                preferred_element_type=jnp.float32,
            )
            acc_v = acc_v + jnp.max(g, axis=0)

    out_ref[0, 0, 0] = total - 2.0 * jnp.sum(acc_v)


@jax.jit
def kernel(x, embedding):
    B, P, T = x.shape
    K, D = embedding.shape
    xp = jnp.pad(x, ((0, 0), (0, 0), (_PAD, _PAD)))
    et = jnp.transpose(embedding.reshape(K, P, _WIN), (2, 1, 0)).reshape(D, K)

    NB = 4
    body = functools.partial(_vq_loss_body, Tlen=T, tblk=2048, nb=NB)
    partials = pl.pallas_call(
        body,
        grid=(B // NB,),
        in_specs=[
            pl.BlockSpec((NB, P, T + 2 * _PAD), lambda b: (b, 0, 0)),
            pl.BlockSpec((D, K), lambda b: (0, 0)),
        ],
        out_specs=pl.BlockSpec((1, 1, 1), lambda b: (b, 0, 0), memory_space=pltpu.SMEM),
        out_shape=jax.ShapeDtypeStruct((B // NB, 1, 1), jnp.float32),
    )(xp, et)
    total = jnp.sum(partials)
    return 0.25 * total / (B * T * D)


# speedup vs baseline: 1.1512x; 1.0057x over previous
import functools

import jax
import jax.numpy as jnp
from jax.experimental import pallas as pl
from jax.experimental.pallas import tpu as pltpu

_WIN = 5
_PAD = (_WIN - 1) // 2


def _vq_loss_body(x_ref, e_ref, out_ref, *, Tlen, tblk, nb):
    e = e_ref[...]
    e2 = jnp.sum(e * e, axis=0)
    ebf = jnp.concatenate(
        [e, (-0.5 * e2)[None, :]], axis=0
    ).astype(jnp.bfloat16)

    total = jnp.float32(0.0)
    acc_v = jnp.zeros((tblk,), jnp.float32)
    for i in range(nb):
        xb = x_ref[i]

        s_all = jnp.sum(xb * xb)
        c0 = xb[:, _PAD:_PAD + 1]
        c1 = xb[:, _PAD + 1:_PAD + 2]
        c2 = xb[:, Tlen:Tlen + 1]
        c3 = xb[:, Tlen + 1:Tlen + 2]
        corr = (2.0 * jnp.sum(c0 * c0) + jnp.sum(c1 * c1)
                + jnp.sum(c2 * c2) + 2.0 * jnp.sum(c3 * c3))
        total = total + 5.0 * s_all - corr

        xbb = xb.astype(jnp.bfloat16)
        ft = jnp.concatenate(
            [xbb[:, w:w + Tlen] for w in range(_WIN)]
            + [jnp.ones((1, Tlen), jnp.bfloat16)],
            axis=0,
        )

        for t0 in range(0, Tlen, tblk):
            g = jax.lax.dot_general(
                ebf, ft[:, t0:t0 + tblk],
                dimension_numbers=(((0,), (0,)), ((), ())),
                preferred_element_type=jnp.float32,
            )
            acc_v = acc_v + jnp.max(g, axis=0)

    out_ref[0, 0, 0] = total - 2.0 * jnp.sum(acc_v)


@jax.jit
def kernel(x, embedding):
    B, P, T = x.shape
    K, D = embedding.shape
    xp = jnp.pad(x, ((0, 0), (0, 0), (_PAD, _PAD)))
    et = jnp.transpose(embedding.reshape(K, P, _WIN), (2, 1, 0)).reshape(D, K)

    NB = 8
    body = functools.partial(_vq_loss_body, Tlen=T, tblk=2048, nb=NB)
    partials = pl.pallas_call(
        body,
        grid=(B // NB,),
        in_specs=[
            pl.BlockSpec((NB, P, T + 2 * _PAD), lambda b: (b, 0, 0)),
            pl.BlockSpec((D, K), lambda b: (0, 0)),
        ],
        out_specs=pl.BlockSpec((1, 1, 1), lambda b: (b, 0, 0), memory_space=pltpu.SMEM),
        out_shape=jax.ShapeDtypeStruct((B // NB, 1, 1), jnp.float32),
    )(xp, et)
    total = jnp.sum(partials)
    return 0.25 * total / (B * T * D)


# speedup vs baseline: 1.2688x; 1.1022x over previous
import functools

import jax
import jax.numpy as jnp
from jax.experimental import pallas as pl
from jax.experimental.pallas import tpu as pltpu

_WIN = 5
_PAD = (_WIN - 1) // 2


def _vq_loss_body(x_ref, e_ref, out_ref, *, Tlen, tblk, nb, nsteps, scale):
    b = pl.program_id(0)
    ebf = e_ref[...]
    P = x_ref.shape[1]

    total = jnp.float32(0.0)
    acc_v = jnp.zeros((tblk,), jnp.float32)
    for i in range(nb):
        xb = x_ref[i]

        s_all = jnp.sum(xb * xb)
        c0 = xb[:, 0:1]
        c1 = xb[:, 1:2]
        c2 = xb[:, Tlen - 2:Tlen - 1]
        c3 = xb[:, Tlen - 1:Tlen]
        corr = (2.0 * jnp.sum(c0 * c0) + jnp.sum(c1 * c1)
                + jnp.sum(c2 * c2) + 2.0 * jnp.sum(c3 * c3))
        total = total + 5.0 * s_all - corr

        xbb = xb.astype(jnp.bfloat16)
        z1 = jnp.zeros((P, 1), jnp.bfloat16)
        z2 = jnp.zeros((P, 2), jnp.bfloat16)
        ft = jnp.concatenate(
            [
                jnp.concatenate([z2, xbb[:, :Tlen - 2]], axis=1),
                jnp.concatenate([z1, xbb[:, :Tlen - 1]], axis=1),
                xbb,
                jnp.concatenate([xbb[:, 1:], z1], axis=1),
                jnp.concatenate([xbb[:, 2:], z2], axis=1),
                jnp.ones((1, Tlen), jnp.bfloat16),
            ],
            axis=0,
        )

        for t0 in range(0, Tlen, tblk):
            g = jax.lax.dot_general(
                ebf, ft[:, t0:t0 + tblk],
                dimension_numbers=(((0,), (0,)), ((), ())),
                preferred_element_type=jnp.float32,
            )
            acc_v = acc_v + jnp.max(g, axis=0)

    total = total - 2.0 * jnp.sum(acc_v)

    @pl.when(b == 0)
    def _init():
        out_ref[0, 0, 0] = 0.0

    out_ref[0, 0, 0] += total

    @pl.when(b == nsteps - 1)
    def _finish():
        out_ref[0, 0, 0] = out_ref[0, 0, 0] * scale


@jax.jit
def kernel(x, embedding):
    B, P, T = x.shape
    K, D = embedding.shape
    et = jnp.transpose(embedding.reshape(K, P, _WIN), (2, 1, 0)).reshape(D, K)
    e2 = jnp.sum(embedding * embedding, axis=1)
    ebf = jnp.concatenate([et, (-0.5 * e2)[None, :]], axis=0).astype(jnp.bfloat16)

    NB = 8
    nsteps = B // NB
    body = functools.partial(
        _vq_loss_body, Tlen=T, tblk=2048, nb=NB, nsteps=nsteps,
        scale=0.25 / (B * T * D),
    )
    out = pl.pallas_call(
        body,
        grid=(nsteps,),
        in_specs=[
            pl.BlockSpec((NB, P, T), lambda b: (b, 0, 0)),
            pl.BlockSpec((D + 1, K), lambda b: (0, 0)),
        ],
        out_specs=pl.BlockSpec((1, 1, 1), lambda b: (0, 0, 0),
                               memory_space=pltpu.SMEM),
        out_shape=jax.ShapeDtypeStruct((1, 1, 1), jnp.float32),
    )(x, ebf)
    return out.reshape(())
